# baseline (speedup 1.0000x reference)
import functools

import jax
import jax.numpy as jnp
from jax import lax
from jax.experimental import pallas as pl
from jax.experimental.pallas import tpu as pltpu
from jax.experimental.pallas import tpu_sc as plsc

L = 16


def _sc_conv(x_flat, cu_pad, w, n_starts):
    T, D = x_flat.shape
    W = w.shape[0]
    info = plsc.get_sparse_core_info()
    NC, NS = info.num_cores, info.num_subcores
    NW = NC * NS
    TPW = T // NW
    TILE = 32
    NT = TPW // TILE
    CG = D // L
    CU = cu_pad.shape[0]
    H = 8

    mesh = plsc.VectorSubcoreMesh(core_axis_name="c", subcore_axis_name="s")

    @functools.partial(
        pl.kernel,
        mesh=mesh,
        out_type=jax.ShapeDtypeStruct((T, D), jnp.float32),
        scratch_types=[
            pltpu.VMEM((TILE + H, D), jnp.float32),
            pltpu.VMEM((TILE + H, D), jnp.float32),
            pltpu.VMEM((TILE, D), jnp.float32),
            pltpu.VMEM((W, D), jnp.float32),
            pltpu.VMEM((CU,), jnp.int32),
            pltpu.SemaphoreType.DMA,
            pltpu.SemaphoreType.DMA,
            pltpu.SemaphoreType.DMA,
        ],
    )
    def k(x_hbm, cu_hbm, w_hbm, out_hbm,
          xb0, xb1, yb, wv, cus, si0, si1, so):
        wid = lax.axis_index("s") * NC + lax.axis_index("c")
        base = wid * TPW
        xbufs = (xb0, xb1)
        sis = (si0, si1)

        pltpu.sync_copy(w_hbm, wv)
        pltpu.sync_copy(cu_hbm, cus)

        def sread(ref, i):
            return ref[pl.ds(i, L)][0]

        @pl.when(wid == 0)
        def _():
            zero = jnp.zeros((L,), jnp.float32)

            def zbody(i, _):
                xb0[H - 3 + i // CG, pl.ds((i % CG) * L, L)] = zero
                return 0

            lax.fori_loop(0, 3 * CG, zbody, 0)

        @pl.when(wid > 0)
        def _():
            pltpu.sync_copy(x_hbm.at[pl.ds(base - H, H)], xb0.at[pl.ds(0, H)])

        pltpu.async_copy(x_hbm.at[pl.ds(base, TILE)],
                         xb0.at[pl.ds(H, TILE)], si0)

        def pair_body(i2, _):
            for b in range(2):
                it = i2 * 2 + b
                start = base + it * TILE
                xb = xbufs[b]
                xbn = xbufs[1 - b]

                @pl.when(it + 1 < NT)
                def _():
                    pltpu.async_copy(
                        x_hbm.at[pl.ds(start + TILE, TILE)],
                        xbn.at[pl.ds(H, TILE)], sis[1 - b])

                pltpu.make_async_copy(
                    x_hbm.at[pl.ds(start, TILE)],
                    xb.at[pl.ds(H, TILE)], sis[b]).wait()

                @pl.when(it + 1 < NT)
                def _():
                    @plsc.parallel_loop(0, CG, 1)
                    def tailcp(cg):
                        col = cg * L
                        for r in range(3):
                            xbn[H - 3 + r, pl.ds(col, L)] = (
                                xb[H + TILE - 3 + r, pl.ds(col, L)])

                @pl.when(it >= 1)
                def _():
                    pltpu.make_async_copy(
                        yb, out_hbm.at[pl.ds(start - TILE, TILE)],
                        so).wait()

                @plsc.parallel_loop(0, CG, 1)
                def cg_body(cg):
                    col = cg * L
                    w0 = wv[0, pl.ds(col, L)]
                    w1 = wv[1, pl.ds(col, L)]
                    w2 = wv[2, pl.ds(col, L)]
                    w3 = wv[3, pl.ds(col, L)]
                    x0 = xb[H - 3, pl.ds(col, L)]
                    x1 = xb[H - 2, pl.ds(col, L)]
                    x2 = xb[H - 1, pl.ds(col, L)]
                    for t in range(TILE):
                        x3 = xb[t + H, pl.ds(col, L)]
                        acc = (x0 * w0 + x1 * w1) + (x2 * w2 + x3 * w3)
                        yb[t, pl.ds(col, L)] = acc / (1.0 + jnp.exp(-acc))
                        x0, x1, x2 = x1, x2, x3

                def fix_body(bi, _):
                    cval = sread(cus, bi)
                    for dt in range(W - 1):
                        t = cval + dt
                        pred = (t >= start) & (t < start + TILE)

                        @pl.when(pred)
                        def _():
                            def dcount(i, dc):
                                ci = sread(cus, i)
                                ok = (ci <= t) & (ci != sread(cus, i - 1))
                                return dc + jnp.where(ok, 1, 0)

                            d = lax.fori_loop(1, n_starts, dcount,
                                              jnp.int32(1))
                            s = sread(cus, d - 1)
                            m = [jnp.where(t - (W - 1) + j >= s,
                                           1.0, 0.0).astype(jnp.float32)
                                 for j in range(W)]
                            row = t - start

                            def cg_fix(cg, _):
                                col = cg * L
                                r0 = row + H - 3
                                acc = ((xb[r0 + 0, pl.ds(col, L)]
                                        * wv[0, pl.ds(col, L)]) * m[0]
                                       + (xb[r0 + 1, pl.ds(col, L)]
                                          * wv[1, pl.ds(col, L)]) * m[1]
                                       + (xb[r0 + 2, pl.ds(col, L)]
                                          * wv[2, pl.ds(col, L)]) * m[2]
                                       + (xb[r0 + 3, pl.ds(col, L)]
                                          * wv[3, pl.ds(col, L)]) * m[3])
                                yb[row, pl.ds(col, L)] = (
                                    acc / (1.0 + jnp.exp(-acc)))
                                return 0

                            lax.fori_loop(0, CG, cg_fix, 0)
                    return 0

                lax.fori_loop(1, n_starts, fix_body, 0)

                pltpu.async_copy(yb, out_hbm.at[pl.ds(start, TILE)], so)
            return 0

        lax.fori_loop(0, NT // 2, pair_body, 0)

        pltpu.make_async_copy(
            yb, out_hbm.at[pl.ds(base + (NT - 1) * TILE, TILE)], so).wait()

    return k(x_flat, cu_pad, w)


def kernel(x, cu_seqlens, kernel):
    B, T, D = x.shape
    W = kernel.shape[0]
    x_flat = x[0]
    w = kernel.reshape(W, D).astype(jnp.float32)
    n_starts = cu_seqlens.shape[0] - 1
    CU = 32
    cu_pad = jnp.concatenate(
        [cu_seqlens.astype(jnp.int32),
         jnp.full((CU - cu_seqlens.shape[0],), T, dtype=jnp.int32)])
    y = _sc_conv(x_flat.astype(jnp.float32), cu_pad, w, n_starts)
    return y.astype(x.dtype)[None]


# speedup vs baseline: 1.2159x; 1.2159x over previous
import functools

import jax
import jax.numpy as jnp
from jax import lax
from jax.experimental import pallas as pl
from jax.experimental.pallas import tpu as pltpu
from jax.experimental.pallas import tpu_sc as plsc

L = 16


def _sc_conv(x_flat, cu_pad, w, n_starts):
    T, D = x_flat.shape
    W = w.shape[0]
    info = plsc.get_sparse_core_info()
    NC, NS = info.num_cores, info.num_subcores
    NW = NC * NS
    TPW = T // NW
    TILE = 16
    NT = TPW // TILE
    CG = D // L
    CU = cu_pad.shape[0]
    H = 8

    mesh = plsc.VectorSubcoreMesh(core_axis_name="c", subcore_axis_name="s")

    @functools.partial(
        pl.kernel,
        mesh=mesh,
        out_type=jax.ShapeDtypeStruct((T, D), jnp.float32),
        scratch_types=[
            pltpu.VMEM((TILE + H, D), jnp.float32),
            pltpu.VMEM((TILE + H, D), jnp.float32),
            pltpu.VMEM((TILE, D), jnp.float32),
            pltpu.VMEM((TILE, D), jnp.float32),
            pltpu.VMEM((W, D), jnp.float32),
            pltpu.VMEM((CU,), jnp.int32),
            pltpu.SemaphoreType.DMA,
            pltpu.SemaphoreType.DMA,
            pltpu.SemaphoreType.DMA,
            pltpu.SemaphoreType.DMA,
        ],
    )
    def k(x_hbm, cu_hbm, w_hbm, out_hbm,
          xb0, xb1, yb0, yb1, wv, cus, si0, si1, so0, so1):
        wid = lax.axis_index("s") * NC + lax.axis_index("c")
        base = wid * TPW
        xbufs, ybufs = (xb0, xb1), (yb0, yb1)
        sis, sos = (si0, si1), (so0, so1)

        pltpu.sync_copy(w_hbm, wv)
        pltpu.sync_copy(cu_hbm, cus)

        def sread(ref, i):
            return ref[pl.ds(i, L)][0]

        @pl.when(wid == 0)
        def _():
            zero = jnp.zeros((L,), jnp.float32)

            def zbody(i, _):
                xb0[H - 3 + i // CG, pl.ds((i % CG) * L, L)] = zero
                return 0

            lax.fori_loop(0, 3 * CG, zbody, 0)

        @pl.when(wid > 0)
        def _():
            pltpu.sync_copy(x_hbm.at[pl.ds(base - H, H)], xb0.at[pl.ds(0, H)])

        pltpu.async_copy(x_hbm.at[pl.ds(base, TILE)],
                         xb0.at[pl.ds(H, TILE)], si0)

        def bmask(bi, msk):
            c = sread(cus, bi)
            for dt in range(W - 1):
                rel = c + dt - base
                ok = (rel >= 0) & (rel < TPW)
                msk = msk | jnp.where(ok, jnp.int32(1) << (rel // TILE),
                                      jnp.int32(0))
            return msk

        fixmask = lax.fori_loop(1, n_starts, bmask, jnp.int32(0))

        def pair_body(i2, _):
            for b in range(2):
                it = i2 * 2 + b
                start = base + it * TILE
                xb, yb = xbufs[b], ybufs[b]
                xbn = xbufs[1 - b]

                @pl.when(it + 1 < NT)
                def _():
                    pltpu.async_copy(
                        x_hbm.at[pl.ds(start + TILE, TILE)],
                        xbn.at[pl.ds(H, TILE)], sis[1 - b])

                pltpu.make_async_copy(
                    x_hbm.at[pl.ds(start, TILE)],
                    xb.at[pl.ds(H, TILE)], sis[b]).wait()

                @pl.when(it >= 2)
                def _():
                    pltpu.make_async_copy(
                        yb, out_hbm.at[pl.ds(start - 2 * TILE, TILE)],
                        sos[b]).wait()

                @plsc.parallel_loop(0, CG, 1)
                def cg_body(cg):
                    col = cg * L
                    w0 = wv[0, pl.ds(col, L)]
                    w1 = wv[1, pl.ds(col, L)]
                    w2 = wv[2, pl.ds(col, L)]
                    w3 = wv[3, pl.ds(col, L)]
                    x0 = xb[H - 3, pl.ds(col, L)]
                    x1 = xb[H - 2, pl.ds(col, L)]
                    x2 = xb[H - 1, pl.ds(col, L)]
                    for t in range(TILE):
                        x3 = xb[t + H, pl.ds(col, L)]
                        acc = (x0 * w0 + x1 * w1) + (x2 * w2 + x3 * w3)
                        yb[t, pl.ds(col, L)] = acc / (1.0 + jnp.exp(-acc))
                        x0, x1, x2 = x1, x2, x3

                    @pl.when(it + 1 < NT)
                    def _():
                        xbn[H - 3, pl.ds(col, L)] = x0
                        xbn[H - 2, pl.ds(col, L)] = x1
                        xbn[H - 1, pl.ds(col, L)] = x2

                def fix_body(bi, _):
                    cval = sread(cus, bi)
                    for dt in range(W - 1):
                        t = cval + dt
                        pred = (t >= start) & (t < start + TILE)

                        @pl.when(pred)
                        def _():
                            def dcount(i, dc):
                                ci = sread(cus, i)
                                ok = (ci <= t) & (ci != sread(cus, i - 1))
                                return dc + jnp.where(ok, 1, 0)

                            d = lax.fori_loop(1, n_starts, dcount,
                                              jnp.int32(1))
                            s = sread(cus, d - 1)
                            m = [jnp.where(t - (W - 1) + j >= s,
                                           1.0, 0.0).astype(jnp.float32)
                                 for j in range(W)]
                            row = t - start

                            def cg_fix(cg, _):
                                col = cg * L
                                r0 = row + H - 3
                                acc = ((xb[r0 + 0, pl.ds(col, L)]
                                        * wv[0, pl.ds(col, L)]) * m[0]
                                       + (xb[r0 + 1, pl.ds(col, L)]
                                          * wv[1, pl.ds(col, L)]) * m[1]
                                       + (xb[r0 + 2, pl.ds(col, L)]
                                          * wv[2, pl.ds(col, L)]) * m[2]
                                       + (xb[r0 + 3, pl.ds(col, L)]
                                          * wv[3, pl.ds(col, L)]) * m[3])
                                yb[row, pl.ds(col, L)] = (
                                    acc / (1.0 + jnp.exp(-acc)))
                                return 0

                            lax.fori_loop(0, CG, cg_fix, 0)
                    return 0

                @pl.when(((fixmask >> it) & 1) != 0)
                def _():
                    lax.fori_loop(1, n_starts, fix_body, 0)

                pltpu.async_copy(yb, out_hbm.at[pl.ds(start, TILE)], sos[b])
            return 0

        lax.fori_loop(0, NT // 2, pair_body, 0)

        pltpu.make_async_copy(
            yb0, out_hbm.at[pl.ds(base + (NT - 2) * TILE, TILE)], so0).wait()
        pltpu.make_async_copy(
            yb1, out_hbm.at[pl.ds(base + (NT - 1) * TILE, TILE)], so1).wait()

    return k(x_flat, cu_pad, w)


def kernel(x, cu_seqlens, kernel):
    B, T, D = x.shape
    W = kernel.shape[0]
    x_flat = x[0]
    w = kernel.reshape(W, D).astype(jnp.float32)
    n_starts = cu_seqlens.shape[0] - 1
    CU = 32
    cu_pad = jnp.concatenate(
        [cu_seqlens.astype(jnp.int32),
         jnp.full((CU - cu_seqlens.shape[0],), T, dtype=jnp.int32)])
    y = _sc_conv(x_flat.astype(jnp.float32), cu_pad, w, n_starts)
    return y.astype(x.dtype)[None]


# speedup vs baseline: 1.4975x; 1.2316x over previous
import functools

import jax
import jax.numpy as jnp
from jax import lax
from jax.experimental import pallas as pl
from jax.experimental.pallas import tpu as pltpu
from jax.experimental.pallas import tpu_sc as plsc

L = 16


def _sc_conv(x_flat, cu_pad, w, n_starts):
    T, D = x_flat.shape
    W = w.shape[0]
    info = plsc.get_sparse_core_info()
    NC, NS = info.num_cores, info.num_subcores
    NW = NC * NS
    TPW = T // NW
    TILE = 16
    NT = TPW // TILE
    CG = D // L
    CU = cu_pad.shape[0]
    H = 8

    mesh = plsc.VectorSubcoreMesh(core_axis_name="c", subcore_axis_name="s")

    @functools.partial(
        pl.kernel,
        mesh=mesh,
        out_type=jax.ShapeDtypeStruct((T, D), jnp.float32),
        scratch_types=[
            pltpu.VMEM((TILE + H, D), jnp.float32),
            pltpu.VMEM((TILE + H, D), jnp.float32),
            pltpu.VMEM((TILE, D), jnp.float32),
            pltpu.VMEM((TILE, D), jnp.float32),
            pltpu.VMEM((W, D), jnp.float32),
            pltpu.VMEM((CU,), jnp.int32),
            pltpu.SemaphoreType.DMA,
            pltpu.SemaphoreType.DMA,
            pltpu.SemaphoreType.DMA,
            pltpu.SemaphoreType.DMA,
        ],
    )
    def k(x_hbm, cu_hbm, w_hbm, out_hbm,
          xb0, xb1, yb0, yb1, wv, cus, si0, si1, so0, so1):
        wid = lax.axis_index("s") * NC + lax.axis_index("c")
        base = wid * TPW
        xbufs, ybufs = (xb0, xb1), (yb0, yb1)
        sis, sos = (si0, si1), (so0, so1)

        pltpu.sync_copy(w_hbm, wv)
        pltpu.sync_copy(cu_hbm, cus)

        def sread(ref, i):
            return ref[pl.ds(i, L)][0]

        @pl.when(wid == 0)
        def _():
            zero = jnp.zeros((L,), jnp.float32)

            def zbody(i, _):
                xb0[H - 3 + i // CG, pl.ds((i % CG) * L, L)] = zero
                return 0

            lax.fori_loop(0, 3 * CG, zbody, 0)

        @pl.when(wid > 0)
        def _():
            pltpu.sync_copy(x_hbm.at[pl.ds(base - H, H)], xb0.at[pl.ds(0, H)])

        pltpu.async_copy(x_hbm.at[pl.ds(base, TILE)],
                         xb0.at[pl.ds(H, TILE)], si0)

        def bmask(bi, msk):
            c = sread(cus, bi)
            for dt in range(W - 1):
                rel = c + dt - base
                ok = (rel >= 0) & (rel < TPW)
                msk = msk | jnp.where(ok, jnp.int32(1) << (rel // TILE),
                                      jnp.int32(0))
            return msk

        fixmask = lax.fori_loop(1, n_starts, bmask, jnp.int32(0))

        def pair_body(i2, _):
            for b in range(2):
                it = i2 * 2 + b
                start = base + it * TILE
                xb, yb = xbufs[b], ybufs[b]
                xbn = xbufs[1 - b]

                @pl.when(it + 1 < NT)
                def _():
                    pltpu.async_copy(
                        x_hbm.at[pl.ds(start + TILE, TILE)],
                        xbn.at[pl.ds(H, TILE)], sis[1 - b])

                pltpu.make_async_copy(
                    x_hbm.at[pl.ds(start, TILE)],
                    xb.at[pl.ds(H, TILE)], sis[b]).wait()

                @pl.when(it >= 2)
                def _():
                    pltpu.make_async_copy(
                        yb, out_hbm.at[pl.ds(start - 2 * TILE, TILE)],
                        sos[b]).wait()

                @plsc.parallel_loop(0, CG, 1)
                def cg_body(cg):
                    col = cg * L
                    w0 = wv[0, pl.ds(col, L)]
                    w1 = wv[1, pl.ds(col, L)]
                    w2 = wv[2, pl.ds(col, L)]
                    w3 = wv[3, pl.ds(col, L)]
                    xs = [xb[r, pl.ds(col, L)]
                          for r in range(H - 3, H + TILE)]
                    accs = [(xs[t] * w0 + xs[t + 1] * w1)
                            + (xs[t + 2] * w2 + xs[t + 3] * w3)
                            for t in range(TILE)]
                    es = [jnp.exp(-a) for a in accs]
                    for t in range(TILE):
                        yb[t, pl.ds(col, L)] = accs[t] / (1.0 + es[t])

                    @pl.when(it + 1 < NT)
                    def _():
                        xbn[H - 3, pl.ds(col, L)] = xs[TILE]
                        xbn[H - 2, pl.ds(col, L)] = xs[TILE + 1]
                        xbn[H - 1, pl.ds(col, L)] = xs[TILE + 2]

                def fix_body(bi, _):
                    cval = sread(cus, bi)
                    for dt in range(W - 1):
                        t = cval + dt
                        pred = (t >= start) & (t < start + TILE)

                        @pl.when(pred)
                        def _():
                            def dcount(i, dc):
                                ci = sread(cus, i)
                                ok = (ci <= t) & (ci != sread(cus, i - 1))
                                return dc + jnp.where(ok, 1, 0)

                            d = lax.fori_loop(1, n_starts, dcount,
                                              jnp.int32(1))
                            s = sread(cus, d - 1)
                            m = [jnp.where(t - (W - 1) + j >= s,
                                           1.0, 0.0).astype(jnp.float32)
                                 for j in range(W)]
                            row = t - start

                            def cg_fix(cg, _):
                                col = cg * L
                                r0 = row + H - 3
                                acc = ((xb[r0 + 0, pl.ds(col, L)]
                                        * wv[0, pl.ds(col, L)]) * m[0]
                                       + (xb[r0 + 1, pl.ds(col, L)]
                                          * wv[1, pl.ds(col, L)]) * m[1]
                                       + (xb[r0 + 2, pl.ds(col, L)]
                                          * wv[2, pl.ds(col, L)]) * m[2]
                                       + (xb[r0 + 3, pl.ds(col, L)]
                                          * wv[3, pl.ds(col, L)]) * m[3])
                                yb[row, pl.ds(col, L)] = (
                                    acc / (1.0 + jnp.exp(-acc)))
                                return 0

                            lax.fori_loop(0, CG, cg_fix, 0)
                    return 0

                @pl.when(((fixmask >> it) & 1) != 0)
                def _():
                    lax.fori_loop(1, n_starts, fix_body, 0)

                pltpu.async_copy(yb, out_hbm.at[pl.ds(start, TILE)], sos[b])
            return 0

        lax.fori_loop(0, NT // 2, pair_body, 0)

        pltpu.make_async_copy(
            yb0, out_hbm.at[pl.ds(base + (NT - 2) * TILE, TILE)], so0).wait()
        pltpu.make_async_copy(
            yb1, out_hbm.at[pl.ds(base + (NT - 1) * TILE, TILE)], so1).wait()

    return k(x_flat, cu_pad, w)


def kernel(x, cu_seqlens, kernel):
    B, T, D = x.shape
    W = kernel.shape[0]
    x_flat = x[0]
    w = kernel.reshape(W, D).astype(jnp.float32)
    n_starts = cu_seqlens.shape[0] - 1
    CU = 32
    cu_pad = jnp.concatenate(
        [cu_seqlens.astype(jnp.int32),
         jnp.full((CU - cu_seqlens.shape[0],), T, dtype=jnp.int32)])
    y = _sc_conv(x_flat.astype(jnp.float32), cu_pad, w, n_starts)
    return y.astype(x.dtype)[None]
